# Initial kernel scaffold; baseline (speedup 1.0000x reference)
import jax
import jax.numpy as jnp
from jax.experimental import pallas as pl


def kernel(features, edge_index, W_init, b_init, W_self, W_neigh, b):
    raise NotImplementedError("write your pallas kernel here")


# speedup vs baseline: 2.8441x; 2.8441x over previous
import functools

import jax
import jax.numpy as jnp
from jax import lax
from jax.experimental import pallas as pl
from jax.experimental.pallas import tpu as pltpu
from jax.experimental.pallas import tpu_sc as plsc

N = 10000
D = 128
H = 128
NL = 3
E = 320000

NW = 32
CHUNK = 128
CHUNKS_PER_W = 80
EPAD = NW * CHUNKS_PER_W * CHUNK
NPAD = 10112
DUMMY_ROW = 10008
ROWS_PER_TILE = NPAD // 16

_f32 = jnp.float32


def _agg_body(h_hbm, src_hbm, dst_hbm, agg_out, deg_out,
              src_v, dst_v, rows_v, deg_v, acc_sh):
    c = lax.axis_index("c")
    s = lax.axis_index("s")
    wid = s * 2 + c

    pltpu.sync_copy(src_hbm.at[wid], src_v)
    pltpu.sync_copy(dst_hbm.at[wid], dst_v)

    zeros16 = jnp.zeros((16,), _f32)
    ones16 = jnp.ones((16,), _f32)

    @pl.loop(0, CHUNK)
    def _zrows(i):
        for k in range(8):
            rows_v[i, pl.ds(16 * k, 16)] = zeros16

    @pl.loop(0, NPAD // 16)
    def _zdeg(i):
        deg_v[pl.ds(i * 16, 16)] = zeros16

    base = s * ROWS_PER_TILE
    for k in range(4):
        pltpu.sync_copy(rows_v, acc_sh.at[pl.ds(base + 128 * k, 128)])
    pltpu.sync_copy(rows_v.at[pl.ds(0, ROWS_PER_TILE - 512)],
                    acc_sh.at[pl.ds(base + 512, ROWS_PER_TILE - 512)])
    plsc.subcore_barrier()

    @pl.loop(0, CHUNKS_PER_W)
    def _chunks(j):
        pltpu.sync_copy(h_hbm.at[src_v.at[j]], rows_v)
        pltpu.sync_copy(rows_v, acc_sh.at[dst_v.at[j]], add=True)
        for k in range(8):
            idx16 = dst_v[j, pl.ds(16 * k, 16)]
            plsc.addupdate_scatter(deg_v, [idx16], ones16)

    plsc.subcore_barrier()

    for k in range(4):
        pltpu.sync_copy(acc_sh.at[pl.ds(base + 128 * k, 128)],
                        agg_out.at[c, pl.ds(base + 128 * k, 128)])
    pltpu.sync_copy(acc_sh.at[pl.ds(base + 512, ROWS_PER_TILE - 512)],
                    agg_out.at[c, pl.ds(base + 512, ROWS_PER_TILE - 512)])
    pltpu.sync_copy(deg_v, deg_out.at[wid])


_agg_call = pl.kernel(
    _agg_body,
    out_type=(jax.ShapeDtypeStruct((2, NPAD, H), _f32),
              jax.ShapeDtypeStruct((NW, NPAD), _f32)),
    mesh=plsc.VectorSubcoreMesh(core_axis_name="c", subcore_axis_name="s"),
    scratch_types=[
        pltpu.VMEM((CHUNKS_PER_W, CHUNK), jnp.int32),
        pltpu.VMEM((CHUNKS_PER_W, CHUNK), jnp.int32),
        pltpu.VMEM((CHUNK, H), _f32),
        pltpu.VMEM((NPAD,), _f32),
        pltpu.VMEM_SHARED((NPAD, H), _f32),
    ],
    compiler_params=pltpu.CompilerParams(needs_layout_passes=False),
)


def _init_body(f_ref, w_ref, b_ref, o_ref):
    o_ref[...] = jnp.maximum(
        jnp.dot(f_ref[...], w_ref[...], preferred_element_type=_f32)
        + b_ref[...], 0.0)


def _rdeg_body(parts_ref, o_ref):
    s = jnp.sum(parts_ref[...], axis=0)
    o_ref[...] = (1.0 / jnp.maximum(s, 1.0))[None, :]


def _layer_body(h_ref, p_ref, r_ref, ws_ref, wn_ref, b_ref, o_ref, *, relu):
    hn = (p_ref[0] + p_ref[1]) * r_ref[...]
    o = (jnp.dot(h_ref[...], ws_ref[...], preferred_element_type=_f32)
         + jnp.dot(hn, wn_ref[...], preferred_element_type=_f32)
         + b_ref[...])
    o_ref[...] = jnp.maximum(o, 0.0) if relu else o


_BR = 1000
_GRID = N // _BR


def _tc_init(features, W_init, b_init):
    return pl.pallas_call(
        _init_body,
        grid=(_GRID,),
        in_specs=[
            pl.BlockSpec((_BR, D), lambda i: (i, 0)),
            pl.BlockSpec((D, H), lambda i: (0, 0)),
            pl.BlockSpec((1, H), lambda i: (0, 0)),
        ],
        out_specs=pl.BlockSpec((_BR, H), lambda i: (i, 0)),
        out_shape=jax.ShapeDtypeStruct((N, H), _f32),
    )(features, W_init, b_init.reshape(1, H))


def _tc_rdeg(deg_parts):
    return pl.pallas_call(
        _rdeg_body,
        in_specs=[pl.BlockSpec((NW, NPAD), lambda: (0, 0))],
        out_specs=pl.BlockSpec((1, NPAD), lambda: (0, 0)),
        out_shape=jax.ShapeDtypeStruct((1, NPAD), _f32),
    )(deg_parts)


def _tc_layer(h, agg_parts, rdeg_b, Ws, Wn, bias, relu):
    return pl.pallas_call(
        functools.partial(_layer_body, relu=relu),
        grid=(_GRID,),
        in_specs=[
            pl.BlockSpec((_BR, H), lambda i: (i, 0)),
            pl.BlockSpec((2, _BR, H), lambda i: (0, i, 0)),
            pl.BlockSpec((_BR, H), lambda i: (i, 0)),
            pl.BlockSpec((H, H), lambda i: (0, 0)),
            pl.BlockSpec((H, H), lambda i: (0, 0)),
            pl.BlockSpec((1, H), lambda i: (0, 0)),
        ],
        out_specs=pl.BlockSpec((_BR, H), lambda i: (i, 0)),
        out_shape=jax.ShapeDtypeStruct((N, H), _f32),
    )(h, agg_parts, rdeg_b, Ws, Wn, bias.reshape(1, H))


def kernel(features, edge_index, W_init, b_init, W_self, W_neigh, b):
    src = edge_index[0]
    dst = edge_index[1]
    pad = EPAD - E
    src_r = jnp.concatenate(
        [src, jnp.zeros((pad,), jnp.int32)]).reshape(NW, CHUNKS_PER_W, CHUNK)
    dst_r = jnp.concatenate(
        [dst, jnp.full((pad,), DUMMY_ROW, jnp.int32)]
    ).reshape(NW, CHUNKS_PER_W, CHUNK)

    h = _tc_init(features, W_init, b_init)

    rdeg_b = None
    for l in range(NL):
        agg_parts, deg_parts = _agg_call(h, src_r, dst_r)
        if rdeg_b is None:
            rdeg = _tc_rdeg(deg_parts)
            rdeg_b = jnp.broadcast_to(rdeg.reshape(NPAD, 1)[:N], (N, H))
        h = _tc_layer(h, agg_parts, rdeg_b, W_self[l], W_neigh[l], b[l],
                      relu=(l < NL - 1))
    return h


# speedup vs baseline: 3.1858x; 1.1201x over previous
import functools

import jax
import jax.numpy as jnp
from jax import lax
from jax.experimental import pallas as pl
from jax.experimental.pallas import tpu as pltpu
from jax.experimental.pallas import tpu_sc as plsc

N = 10000
D = 128
H = 128
NL = 3
E = 320000

NW = 32
CHUNK = 64
STAGES = 2
CHUNKS_PER_STAGE = 80
CHUNKS_PER_W = STAGES * CHUNKS_PER_STAGE
EPAD = NW * CHUNKS_PER_W * CHUNK
NPAD = 10112
DUMMY_ROW = 10008
ROWS_PER_TILE = NPAD // 16

_f32 = jnp.float32


def _agg_body(h_hbm, src_hbm, dst_hbm, agg_out, deg_out,
              src_v, dst_v, rows0, rows1, deg_v, acc_sh,
              gsem0, gsem1, ssem0, ssem1):
    c = lax.axis_index("c")
    s = lax.axis_index("s")
    wid = s * 2 + c

    zeros16 = jnp.zeros((16,), _f32)
    ones16 = jnp.ones((16,), _f32)

    @pl.loop(0, CHUNK)
    def _zrows(i):
        for k in range(8):
            rows0[i, pl.ds(16 * k, 16)] = zeros16

    @pl.loop(0, NPAD // 16)
    def _zdeg(i):
        deg_v[pl.ds(i * 16, 16)] = zeros16

    base = s * ROWS_PER_TILE
    for k in range(9):
        pltpu.sync_copy(rows0, acc_sh.at[pl.ds(base + CHUNK * k, CHUNK)])
    rem = ROWS_PER_TILE - 9 * CHUNK
    pltpu.sync_copy(rows0.at[pl.ds(0, rem)],
                    acc_sh.at[pl.ds(base + 9 * CHUNK, rem)])
    plsc.subcore_barrier()

    def g_start(j, buf, sem):
        pltpu.async_copy(h_hbm.at[src_v.at[j]], buf, sem)

    def g_wait(buf, sem):
        pltpu.make_async_copy(h_hbm.at[src_v.at[0]], buf, sem).wait()

    def s_start(j, buf, sem):
        pltpu.async_copy(buf, acc_sh.at[dst_v.at[j]], sem, add=True)

    def s_wait(buf, sem):
        pltpu.make_async_copy(buf, acc_sh.at[dst_v.at[0]], sem).wait()

    def deg_add(j):
        for k in range(CHUNK // 16):
            idx16 = dst_v[j, pl.ds(16 * k, 16)]
            plsc.addupdate_scatter(deg_v, [idx16], ones16)

    for stage in range(STAGES):
        pltpu.sync_copy(src_hbm.at[wid, stage], src_v)
        pltpu.sync_copy(dst_hbm.at[wid, stage], dst_v)
        g_start(0, rows0, gsem0)

        @pl.loop(0, CHUNKS_PER_STAGE, step=2)
        def _chunks(j):
            g_wait(rows0, gsem0)

            @pl.when(j > 0)
            def _():
                s_wait(rows1, ssem1)

            g_start(j + 1, rows1, gsem1)
            s_start(j, rows0, ssem0)
            deg_add(j)
            deg_add(j + 1)
            g_wait(rows1, gsem1)
            s_wait(rows0, ssem0)

            @pl.when(j + 2 < CHUNKS_PER_STAGE)
            def _():
                g_start(j + 2, rows0, gsem0)

            s_start(j + 1, rows1, ssem1)

        s_wait(rows1, ssem1)
    plsc.subcore_barrier()

    pltpu.sync_copy(acc_sh.at[pl.ds(base, ROWS_PER_TILE)],
                    agg_out.at[c, pl.ds(base, ROWS_PER_TILE)])
    pltpu.sync_copy(deg_v, deg_out.at[wid])


_agg_call = pl.kernel(
    _agg_body,
    out_type=(jax.ShapeDtypeStruct((2, NPAD, H), _f32),
              jax.ShapeDtypeStruct((NW, NPAD), _f32)),
    mesh=plsc.VectorSubcoreMesh(core_axis_name="c", subcore_axis_name="s"),
    scratch_types=[
        pltpu.VMEM((CHUNKS_PER_STAGE, CHUNK), jnp.int32),
        pltpu.VMEM((CHUNKS_PER_STAGE, CHUNK), jnp.int32),
        pltpu.VMEM((CHUNK, H), _f32),
        pltpu.VMEM((CHUNK, H), _f32),
        pltpu.VMEM((NPAD,), _f32),
        pltpu.VMEM_SHARED((NPAD, H), _f32),
        pltpu.SemaphoreType.DMA,
        pltpu.SemaphoreType.DMA,
        pltpu.SemaphoreType.DMA,
        pltpu.SemaphoreType.DMA,
    ],
    compiler_params=pltpu.CompilerParams(needs_layout_passes=False),
)


def _init_body(f_ref, w_ref, b_ref, o_ref):
    o_ref[...] = jnp.maximum(
        jnp.dot(f_ref[...], w_ref[...], preferred_element_type=_f32)
        + b_ref[...], 0.0)


def _rdeg_body(parts_ref, o_ref):
    s = jnp.sum(parts_ref[...], axis=0)
    o_ref[...] = (1.0 / jnp.maximum(s, 1.0))[None, :]


def _layer_body(h_ref, p_ref, r_ref, ws_ref, wn_ref, b_ref, o_ref, *, relu):
    hn = (p_ref[0] + p_ref[1]) * r_ref[...]
    o = (jnp.dot(h_ref[...], ws_ref[...], preferred_element_type=_f32)
         + jnp.dot(hn, wn_ref[...], preferred_element_type=_f32)
         + b_ref[...])
    o_ref[...] = jnp.maximum(o, 0.0) if relu else o


_BR = 1000
_GRID = N // _BR


def _tc_init(features, W_init, b_init):
    return pl.pallas_call(
        _init_body,
        grid=(_GRID,),
        in_specs=[
            pl.BlockSpec((_BR, D), lambda i: (i, 0)),
            pl.BlockSpec((D, H), lambda i: (0, 0)),
            pl.BlockSpec((1, H), lambda i: (0, 0)),
        ],
        out_specs=pl.BlockSpec((_BR, H), lambda i: (i, 0)),
        out_shape=jax.ShapeDtypeStruct((N, H), _f32),
    )(features, W_init, b_init.reshape(1, H))


def _tc_rdeg(deg_parts):
    return pl.pallas_call(
        _rdeg_body,
        in_specs=[pl.BlockSpec((NW, NPAD), lambda: (0, 0))],
        out_specs=pl.BlockSpec((1, NPAD), lambda: (0, 0)),
        out_shape=jax.ShapeDtypeStruct((1, NPAD), _f32),
    )(deg_parts)


def _tc_layer(h, agg_parts, rdeg_b, Ws, Wn, bias, relu):
    return pl.pallas_call(
        functools.partial(_layer_body, relu=relu),
        grid=(_GRID,),
        in_specs=[
            pl.BlockSpec((_BR, H), lambda i: (i, 0)),
            pl.BlockSpec((2, _BR, H), lambda i: (0, i, 0)),
            pl.BlockSpec((_BR, H), lambda i: (i, 0)),
            pl.BlockSpec((H, H), lambda i: (0, 0)),
            pl.BlockSpec((H, H), lambda i: (0, 0)),
            pl.BlockSpec((1, H), lambda i: (0, 0)),
        ],
        out_specs=pl.BlockSpec((_BR, H), lambda i: (i, 0)),
        out_shape=jax.ShapeDtypeStruct((N, H), _f32),
    )(h, agg_parts, rdeg_b, Ws, Wn, bias.reshape(1, H))


def kernel(features, edge_index, W_init, b_init, W_self, W_neigh, b):
    src = edge_index[0]
    dst = edge_index[1]
    pad = EPAD - E
    src_r = jnp.concatenate(
        [src, jnp.zeros((pad,), jnp.int32)]
    ).reshape(NW, STAGES, CHUNKS_PER_STAGE, CHUNK)
    dst_r = jnp.concatenate(
        [dst, jnp.full((pad,), DUMMY_ROW, jnp.int32)]
    ).reshape(NW, STAGES, CHUNKS_PER_STAGE, CHUNK)

    h = _tc_init(features, W_init, b_init)

    rdeg_b = None
    for l in range(NL):
        agg_parts, deg_parts = _agg_call(h, src_r, dst_r)
        if rdeg_b is None:
            rdeg = _tc_rdeg(deg_parts)
            rdeg_b = jnp.broadcast_to(rdeg.reshape(NPAD, 1)[:N], (N, H))
        h = _tc_layer(h, agg_parts, rdeg_b, W_self[l], W_neigh[l], b[l],
                      relu=(l < NL - 1))
    return h


# speedup vs baseline: 3.3660x; 1.0566x over previous
import functools

import jax
import jax.numpy as jnp
from jax import lax
from jax.experimental import pallas as pl
from jax.experimental.pallas import tpu as pltpu
from jax.experimental.pallas import tpu_sc as plsc

N = 10000
D = 128
H = 128
NL = 3
E = 320000

NW = 32
CHUNK = 64
STAGES = 4
CPS0 = 56
CPS1 = 24
E0 = 16 * STAGES * CPS0 * CHUNK
E1 = 16 * STAGES * CPS1 * CHUNK
EPAD = E0 + E1
NPAD = 10112
DUMMY_ROW = 10008
ROWS_PER_TILE = NPAD // 16

_f32 = jnp.float32


def _agg_body(h_hbm, src0_hbm, dst0_hbm, src1_hbm, dst1_hbm, agg_out,
              deg_out, src_v, dst_v, rows0, rows1, deg_v, acc_sh,
              gsem0, gsem1, ssem0, ssem1):
    c = lax.axis_index("c")
    s = lax.axis_index("s")
    wid = s * 2 + c

    zeros16 = jnp.zeros((16,), _f32)
    ones16 = jnp.ones((16,), _f32)

    @pl.loop(0, CHUNK)
    def _zrows(i):
        for k in range(8):
            rows0[i, pl.ds(16 * k, 16)] = zeros16

    @pl.loop(0, NPAD // 16)
    def _zdeg(i):
        deg_v[pl.ds(i * 16, 16)] = zeros16

    base = s * ROWS_PER_TILE
    for k in range(9):
        pltpu.sync_copy(rows0, acc_sh.at[pl.ds(base + CHUNK * k, CHUNK)])
    rem = ROWS_PER_TILE - 9 * CHUNK
    pltpu.sync_copy(rows0.at[pl.ds(0, rem)],
                    acc_sh.at[pl.ds(base + 9 * CHUNK, rem)])
    plsc.subcore_barrier()

    def g_start(j, buf, sem):
        pltpu.async_copy(h_hbm.at[src_v.at[j]], buf, sem)

    def g_wait(buf, sem):
        pltpu.make_async_copy(h_hbm.at[src_v.at[0]], buf, sem).wait()

    def s_start(j, buf, sem):
        pltpu.async_copy(buf, acc_sh.at[dst_v.at[j]], sem, add=True)

    def s_wait(buf, sem):
        pltpu.make_async_copy(buf, acc_sh.at[dst_v.at[0]], sem).wait()

    def deg_add(j):
        for k in range(CHUNK // 16):
            idx16 = dst_v[j, pl.ds(16 * k, 16)]
            plsc.addupdate_scatter(deg_v, [idx16], ones16)

    def run_pipeline(src_hbm, dst_hbm, cps):
        for stage in range(STAGES):
            pltpu.sync_copy(src_hbm.at[s, stage], src_v.at[pl.ds(0, cps)])
            pltpu.sync_copy(dst_hbm.at[s, stage], dst_v.at[pl.ds(0, cps)])
            g_start(0, rows0, gsem0)

            @pl.loop(0, cps, step=2)
            def _chunks(j):
                g_wait(rows0, gsem0)

                @pl.when(j > 0)
                def _():
                    s_wait(rows1, ssem1)

                g_start(j + 1, rows1, gsem1)
                s_start(j, rows0, ssem0)
                deg_add(j)
                deg_add(j + 1)
                g_wait(rows1, gsem1)
                s_wait(rows0, ssem0)

                @pl.when(j + 2 < cps)
                def _():
                    g_start(j + 2, rows0, gsem0)

                s_start(j + 1, rows1, ssem1)

            s_wait(rows1, ssem1)

    @pl.when(c == 0)
    def _():
        run_pipeline(src0_hbm, dst0_hbm, CPS0)

    @pl.when(c == 1)
    def _():
        run_pipeline(src1_hbm, dst1_hbm, CPS1)

    plsc.subcore_barrier()

    pltpu.sync_copy(acc_sh.at[pl.ds(base, ROWS_PER_TILE)],
                    agg_out.at[c, pl.ds(base, ROWS_PER_TILE)])
    pltpu.sync_copy(deg_v, deg_out.at[wid])


_agg_call = pl.kernel(
    _agg_body,
    out_type=(jax.ShapeDtypeStruct((2, NPAD, H), _f32),
              jax.ShapeDtypeStruct((NW, NPAD), _f32)),
    mesh=plsc.VectorSubcoreMesh(core_axis_name="c", subcore_axis_name="s"),
    scratch_types=[
        pltpu.VMEM((CPS0, CHUNK), jnp.int32),
        pltpu.VMEM((CPS0, CHUNK), jnp.int32),
        pltpu.VMEM((CHUNK, H), _f32),
        pltpu.VMEM((CHUNK, H), _f32),
        pltpu.VMEM((NPAD,), _f32),
        pltpu.VMEM_SHARED((NPAD, H), _f32),
        pltpu.SemaphoreType.DMA,
        pltpu.SemaphoreType.DMA,
        pltpu.SemaphoreType.DMA,
        pltpu.SemaphoreType.DMA,
    ],
    compiler_params=pltpu.CompilerParams(needs_layout_passes=False),
)


def _init_body(f_ref, w_ref, b_ref, o_ref):
    o_ref[...] = jnp.maximum(
        jnp.dot(f_ref[...], w_ref[...], preferred_element_type=_f32)
        + b_ref[...], 0.0)


def _rdeg_body(parts_ref, o_ref):
    s = jnp.sum(parts_ref[...], axis=0)
    o_ref[...] = (1.0 / jnp.maximum(s, 1.0))[None, :]


def _layer_body(h_ref, p_ref, r_ref, ws_ref, wn_ref, b_ref, o_ref, *, relu):
    hn = (p_ref[0] + p_ref[1]) * r_ref[...]
    o = (jnp.dot(h_ref[...], ws_ref[...], preferred_element_type=_f32)
         + jnp.dot(hn, wn_ref[...], preferred_element_type=_f32)
         + b_ref[...])
    o_ref[...] = jnp.maximum(o, 0.0) if relu else o


_BR = 1000
_GRID = N // _BR


def _tc_init(features, W_init, b_init):
    return pl.pallas_call(
        _init_body,
        grid=(_GRID,),
        in_specs=[
            pl.BlockSpec((_BR, D), lambda i: (i, 0)),
            pl.BlockSpec((D, H), lambda i: (0, 0)),
            pl.BlockSpec((1, H), lambda i: (0, 0)),
        ],
        out_specs=pl.BlockSpec((_BR, H), lambda i: (i, 0)),
        out_shape=jax.ShapeDtypeStruct((N, H), _f32),
    )(features, W_init, b_init.reshape(1, H))


def _tc_rdeg(deg_parts):
    return pl.pallas_call(
        _rdeg_body,
        in_specs=[pl.BlockSpec((NW, NPAD), lambda: (0, 0))],
        out_specs=pl.BlockSpec((1, NPAD), lambda: (0, 0)),
        out_shape=jax.ShapeDtypeStruct((1, NPAD), _f32),
    )(deg_parts)


def _tc_layer(h, agg_parts, rdeg_b, Ws, Wn, bias, relu):
    return pl.pallas_call(
        functools.partial(_layer_body, relu=relu),
        grid=(_GRID,),
        in_specs=[
            pl.BlockSpec((_BR, H), lambda i: (i, 0)),
            pl.BlockSpec((2, _BR, H), lambda i: (0, i, 0)),
            pl.BlockSpec((_BR, H), lambda i: (i, 0)),
            pl.BlockSpec((H, H), lambda i: (0, 0)),
            pl.BlockSpec((H, H), lambda i: (0, 0)),
            pl.BlockSpec((1, H), lambda i: (0, 0)),
        ],
        out_specs=pl.BlockSpec((_BR, H), lambda i: (i, 0)),
        out_shape=jax.ShapeDtypeStruct((N, H), _f32),
    )(h, agg_parts, rdeg_b, Ws, Wn, bias.reshape(1, H))


def kernel(features, edge_index, W_init, b_init, W_self, W_neigh, b):
    src = edge_index[0]
    dst = edge_index[1]
    pad = EPAD - E
    src_p = jnp.concatenate([src, jnp.zeros((pad,), jnp.int32)])
    dst_p = jnp.concatenate([dst, jnp.full((pad,), DUMMY_ROW, jnp.int32)])
    src0 = src_p[:E0].reshape(16, STAGES, CPS0, CHUNK)
    dst0 = dst_p[:E0].reshape(16, STAGES, CPS0, CHUNK)
    src1 = src_p[E0:].reshape(16, STAGES, CPS1, CHUNK)
    dst1 = dst_p[E0:].reshape(16, STAGES, CPS1, CHUNK)

    h = _tc_init(features, W_init, b_init)

    rdeg_b = None
    for l in range(NL):
        agg_parts, deg_parts = _agg_call(h, src0, dst0, src1, dst1)
        if rdeg_b is None:
            rdeg = _tc_rdeg(deg_parts)
            rdeg_b = jnp.broadcast_to(rdeg.reshape(NPAD, 1)[:N], (N, H))
        h = _tc_layer(h, agg_parts, rdeg_b, W_self[l], W_neigh[l], b[l],
                      relu=(l < NL - 1))
    return h
